# baseline (speedup 1.0000x reference)
import functools

import jax
import jax.numpy as jnp
from jax import lax
from jax.experimental import pallas as pl
from jax.experimental.pallas import tpu as pltpu
from jax.experimental.pallas import tpu_sc as plsc

B = 4
S = 4096
D = 1024
PAD = 1
NC = 2
NS = 16
L = 16
NW = NC * NS
JW = S // NW
C = 32
NCH = JW // C

_mesh = plsc.VectorSubcoreMesh(core_axis_name="c", subcore_axis_name="s")


@functools.partial(
    pl.kernel,
    out_type=jax.ShapeDtypeStruct((B, S, D), jnp.float32),
    mesh=_mesh,
    compiler_params=pltpu.CompilerParams(use_tc_tiling_on_sc=False, needs_layout_passes=False),
    scratch_types=[
        pltpu.VMEM((B, JW), jnp.int32),
        pltpu.VMEM((2, C, D), jnp.float32),
        pltpu.VMEM((D,), jnp.float32),
    ],
)
def _sinus_embed(x_hbm, w_hbm, out_hbm, xbuf, wbuf, padrow):
    wid = lax.axis_index("s") * NC + lax.axis_index("c")
    j0 = wid * JW

    for b in range(B):
        pltpu.sync_copy(x_hbm.at[b, pl.ds(j0, JW)], xbuf.at[b])
    pltpu.sync_copy(w_hbm.at[PAD], padrow)

    for c in range(NCH):
        jc = j0 + c * C
        buf = c % 2
        pltpu.sync_copy(w_hbm.at[pl.ds(jc + 2, C)], wbuf.at[buf])
        for b in range(B):
            pads = []
            for g in range(C // L):
                xv = xbuf[b, pl.ds(c * C + g * L, L)]
                pads.append(jnp.where(xv == PAD, 1, 0).astype(jnp.int32))
            npad = sum(jnp.sum(p) for p in pads)

            @pl.when(npad == 0)
            def _bulk(b=b, jc=jc, buf=buf):
                pltpu.sync_copy(wbuf.at[buf], out_hbm.at[b, pl.ds(jc, C)])

            @pl.when(npad != 0)
            def _fixup(b=b, jc=jc, buf=buf, pads=pads):
                lane = lax.broadcasted_iota(jnp.int32, (L,), 0)
                for g in range(C // L):
                    padv = pads[g]

                    def row_body(r, _, g=g, padv=padv):
                        is_pad = jnp.sum(jnp.where(lane == r, padv, 0))

                        @pl.when(is_pad != 0)
                        def _pad_row():
                            pltpu.sync_copy(padrow, out_hbm.at[b, jc + g * L + r])

                        @pl.when(is_pad == 0)
                        def _w_row():
                            pltpu.sync_copy(
                                wbuf.at[buf, g * L + r],
                                out_hbm.at[b, jc + g * L + r],
                            )

                        return 0

                    lax.fori_loop(0, L, row_body, 0)


def kernel(X, weights):
    return _sinus_embed(X, weights)


# speedup vs baseline: 1.0101x; 1.0101x over previous
import functools

import jax
import jax.numpy as jnp
from jax import lax
from jax.experimental import pallas as pl
from jax.experimental.pallas import tpu as pltpu
from jax.experimental.pallas import tpu_sc as plsc

B = 4
S = 4096
D = 1024
PAD = 1
NC = 2
NS = 16
L = 16
NW = NC * NS
JW = S // NW
C = 32
NCH = JW // C

_mesh = plsc.VectorSubcoreMesh(core_axis_name="c", subcore_axis_name="s")


@functools.partial(
    pl.kernel,
    out_type=jax.ShapeDtypeStruct((B, S, D), jnp.float32),
    mesh=_mesh,
    compiler_params=pltpu.CompilerParams(use_tc_tiling_on_sc=False, needs_layout_passes=False),
    scratch_types=[
        pltpu.VMEM((B, JW), jnp.int32),
        pltpu.VMEM((2, C, D), jnp.float32),
        pltpu.VMEM((D,), jnp.float32),
        pltpu.SemaphoreType.DMA,
        pltpu.SemaphoreType.DMA,
        pltpu.SemaphoreType.DMA,
    ],
)
def _sinus_embed(x_hbm, w_hbm, out_hbm, xbuf, wbuf, padrow, rsem, wsem0, wsem1):
    wid = lax.axis_index("s") * NC + lax.axis_index("c")
    j0 = wid * JW
    wsems = (wsem0, wsem1)

    for b in range(B):
        pltpu.sync_copy(x_hbm.at[b, pl.ds(j0, JW)], xbuf.at[b])
    pltpu.sync_copy(w_hbm.at[PAD], padrow)

    read_descs = [None] * NCH
    write_descs = [None] * NCH
    read_descs[0] = pltpu.async_copy(
        w_hbm.at[pl.ds(j0 + 2, C)], wbuf.at[0], rsem
    )

    for c in range(NCH):
        buf = c % 2
        read_descs[c].wait()
        if c + 1 < NCH:
            if write_descs[c - 1] is not None:
                for d in write_descs[c - 1]:
                    d.wait()
                write_descs[c - 1] = None
            read_descs[c + 1] = pltpu.async_copy(
                w_hbm.at[pl.ds(j0 + 2 + (c + 1) * C, C)],
                wbuf.at[1 - buf],
                rsem,
            )
        jc = j0 + c * C
        write_descs[c] = [
            pltpu.async_copy(wbuf.at[buf], out_hbm.at[b, pl.ds(jc, C)], wsems[buf])
            for b in range(B)
        ]

    for c in range(NCH):
        if write_descs[c] is not None:
            for d in write_descs[c]:
                d.wait()

    lane = lax.broadcasted_iota(jnp.int32, (L,), 0)
    for b in range(B):
        for g in range(JW // L):
            xv = xbuf[b, pl.ds(g * L, L)]
            padv = jnp.where(xv == PAD, 1, 0).astype(jnp.int32)
            npad = jnp.sum(padv)

            @pl.when(npad != 0)
            def _fixup(b=b, g=g, padv=padv):
                def row_body(r, _):
                    is_pad = jnp.sum(jnp.where(lane == r, padv, 0))

                    @pl.when(is_pad != 0)
                    def _pad_row():
                        pltpu.sync_copy(padrow, out_hbm.at[b, j0 + g * L + r])

                    return 0

                lax.fori_loop(0, L, row_body, 0)


def kernel(X, weights):
    return _sinus_embed(X, weights)


# speedup vs baseline: 2.1951x; 2.1731x over previous
import functools

import jax
import jax.numpy as jnp
from jax import lax
from jax.experimental import pallas as pl
from jax.experimental.pallas import tpu as pltpu
from jax.experimental.pallas import tpu_sc as plsc

B = 4
S = 4096
D = 1024
PAD = 1
NC = 2
NS = 16
L = 16
NW = NC * NS
JW = S // NW
C = 32
NCH = JW // C
G = JW // L

_mesh = plsc.VectorSubcoreMesh(core_axis_name="c", subcore_axis_name="s")


@functools.partial(
    pl.kernel,
    out_type=jax.ShapeDtypeStruct((B, S, D), jnp.float32),
    mesh=_mesh,
    compiler_params=pltpu.CompilerParams(needs_layout_passes=False),
    scratch_types=[
        pltpu.VMEM((B * JW,), jnp.int32),
        pltpu.VMEM((2, C, D), jnp.float32),
        pltpu.VMEM((D,), jnp.float32),
        pltpu.VMEM((L, D), jnp.float32),
        pltpu.SemaphoreType.DMA,
        pltpu.SemaphoreType.DMA,
        pltpu.SemaphoreType.DMA,
    ],
)
def _sinus_embed(x_hbm, wsh_hbm, pad_hbm, out_hbm, xbuf, wbuf, padbuf, tbuf,
                 rsem, wsem0, wsem1):
    wid = lax.axis_index("s") * NC + lax.axis_index("c")
    j0 = wid * JW
    wsems = (wsem0, wsem1)

    for b in range(B):
        pltpu.sync_copy(x_hbm.at[pl.ds(b * S + j0, JW)],
                        xbuf.at[pl.ds(b * JW, JW)])
    pltpu.sync_copy(pad_hbm, padbuf)

    read_descs = [None] * NCH
    write_descs = [None] * NCH
    read_descs[0] = pltpu.async_copy(wsh_hbm.at[pl.ds(j0, C)], wbuf.at[0], rsem)

    for c in range(NCH):
        buf = c % 2
        read_descs[c].wait()
        if c + 1 < NCH:
            if c >= 1:
                for d in write_descs[c - 1]:
                    d.wait()
                write_descs[c - 1] = None
            read_descs[c + 1] = pltpu.async_copy(
                wsh_hbm.at[pl.ds(j0 + (c + 1) * C, C)], wbuf.at[1 - buf], rsem
            )
        jc = j0 + c * C
        write_descs[c] = [
            pltpu.async_copy(wbuf.at[buf], out_hbm.at[b, pl.ds(jc, C)], wsems[buf])
            for b in range(B)
        ]

    for c in range(NCH):
        if write_descs[c] is not None:
            for d in write_descs[c]:
                d.wait()

    lane = lax.broadcasted_iota(jnp.int32, (L,), 0)
    gpb = JW // L

    def group_body(i, _):
        b = i // gpb
        g = i - b * gpb
        jg = j0 + g * L
        xv = xbuf[pl.ds(i * L, L)]
        padv = jnp.where(xv == PAD, 1, 0).astype(jnp.int32)
        npad = jnp.sum(padv)

        @pl.when(npad != 0)
        def _fix():
            pltpu.sync_copy(wsh_hbm.at[pl.ds(jg, L)], tbuf)
            for r in range(L):
                is_pad = jnp.sum(jnp.where(lane == r, padv, 0))

                @pl.when(is_pad != 0)
                def _patch(r=r):
                    for dd in range(D // L):
                        tbuf[r, pl.ds(dd * L, L)] = padbuf[pl.ds(dd * L, L)]

            pltpu.sync_copy(tbuf, out_hbm.at[b, pl.ds(jg, L)])

        return 0

    lax.fori_loop(0, B * gpb, group_body, 0)


def kernel(X, weights):
    wsh = lax.slice(weights, (2, 0), (2 + S, D))
    pad_row = lax.slice(weights, (PAD, 0), (PAD + 1, D)).reshape(D)
    return _sinus_embed(X.reshape(-1), wsh, pad_row)


# speedup vs baseline: 2.5431x; 1.1585x over previous
import functools

import jax
import jax.numpy as jnp
from jax import lax
from jax.experimental import pallas as pl
from jax.experimental.pallas import tpu as pltpu
from jax.experimental.pallas import tpu_sc as plsc

B = 4
S = 4096
D = 1024
PAD = 1
NC = 2
NS = 16
L = 16
NW = NC * NS
JW = S // NW
C = 32
NCH = JW // C

_mesh = plsc.VectorSubcoreMesh(core_axis_name="c", subcore_axis_name="s")


@functools.partial(
    pl.kernel,
    out_type=jax.ShapeDtypeStruct((B, S, D), jnp.float32),
    mesh=_mesh,
    compiler_params=pltpu.CompilerParams(needs_layout_passes=False),
    scratch_types=[
        pltpu.VMEM((B, JW), jnp.int32),
        pltpu.VMEM((JW,), jnp.int32),
        pltpu.VMEM((2, C, D), jnp.float32),
        pltpu.VMEM((8, D), jnp.float32),
        pltpu.VMEM((L, D), jnp.float32),
        pltpu.SemaphoreType.DMA,
        pltpu.SemaphoreType.DMA,
        pltpu.SemaphoreType.DMA,
    ],
)
def _sinus_embed(x_hbm, w_hbm, idx_hbm, out_hbm, xbuf, idxvm, wbuf, padbuf,
                 tbuf, rsem, wsem0, wsem1):
    wid = lax.axis_index("s") * NC + lax.axis_index("c")
    j0 = wid * JW
    wsems = (wsem0, wsem1)

    pltpu.sync_copy(x_hbm.at[:, pl.ds(j0, JW)], xbuf)
    pltpu.sync_copy(idx_hbm.at[pl.ds(j0, JW)], idxvm)
    pltpu.sync_copy(w_hbm.at[pl.ds(0, 8)], padbuf)

    read_descs = [None] * NCH
    write_descs = [None] * NCH
    read_descs[0] = pltpu.async_copy(
        w_hbm.at[idxvm.at[pl.ds(0, C)]], wbuf.at[0], rsem
    )

    for c in range(NCH):
        buf = c % 2
        read_descs[c].wait()
        if c + 1 < NCH:
            if c >= 1:
                for d in write_descs[c - 1]:
                    d.wait()
                write_descs[c - 1] = None
            read_descs[c + 1] = pltpu.async_copy(
                w_hbm.at[idxvm.at[pl.ds((c + 1) * C, C)]], wbuf.at[1 - buf], rsem
            )
        jc = j0 + c * C
        write_descs[c] = [
            pltpu.async_copy(
                wbuf.at[buf], out_hbm.at[b, pl.ds(jc, C)], wsems[buf]
            )
            for b in range(B)
        ]

    for descs in write_descs:
        if descs is not None:
            for d in descs:
                d.wait()

    lane = lax.broadcasted_iota(jnp.int32, (L,), 0)
    gpb = JW // L

    for b in range(B):

        def group_body(g, _, b=b):
            jg = j0 + g * L
            xv = xbuf[b, pl.ds(g * L, L)]
            padv = jnp.where(xv == PAD, 1, 0).astype(jnp.int32)
            npad = jnp.sum(padv)

            @pl.when(npad != 0)
            def _fix():
                pltpu.sync_copy(out_hbm.at[b, pl.ds(jg, L)], tbuf)
                for r in range(L):
                    is_pad = jnp.sum(jnp.where(lane == r, padv, 0))

                    @pl.when(is_pad != 0)
                    def _patch(r=r):
                        def d_body(dd, _):
                            tbuf[r, pl.ds(dd * L, L)] = padbuf[PAD, pl.ds(dd * L, L)]
                            return 0

                        lax.fori_loop(0, D // L, d_body, 0)

                pltpu.sync_copy(tbuf, out_hbm.at[b, pl.ds(jg, L)])

            return 0

        lax.fori_loop(0, gpb, group_body, 0)


def kernel(X, weights):
    idx = jnp.arange(2, S + 2, dtype=jnp.int32)
    return _sinus_embed(X, weights, idx)


# speedup vs baseline: 2.6851x; 1.0558x over previous
import functools

import jax
import jax.numpy as jnp
from jax import lax
from jax.experimental import pallas as pl
from jax.experimental.pallas import tpu as pltpu
from jax.experimental.pallas import tpu_sc as plsc

B = 4
S = 4096
D = 1024
PAD = 1
NC = 2
NS = 16
L = 16
NW = NC * NS
JW = S // NW
C = 32
NCH = JW // C

_mesh = plsc.VectorSubcoreMesh(core_axis_name="c", subcore_axis_name="s")


@functools.partial(
    pl.kernel,
    out_type=jax.ShapeDtypeStruct((B, S, D), jnp.float32),
    mesh=_mesh,
    compiler_params=pltpu.CompilerParams(needs_layout_passes=False),
    scratch_types=[
        pltpu.VMEM((B, JW), jnp.int32),
        pltpu.VMEM((JW,), jnp.int32),
        pltpu.VMEM((2, C, D), jnp.float32),
        pltpu.VMEM((8, D), jnp.float32),
        pltpu.VMEM((L, D), jnp.float32),
        pltpu.SemaphoreType.DMA,
        pltpu.SemaphoreType.DMA,
        pltpu.SemaphoreType.DMA,
        pltpu.SemaphoreType.DMA,
    ],
)
def _sinus_embed(x_hbm, w_hbm, idx_hbm, out_hbm, xbuf, idxvm, wbuf, padbuf,
                 tbuf, rsem, wsem0, wsem1, ssem):
    wid = lax.axis_index("s") * NC + lax.axis_index("c")
    j0 = wid * JW
    wsems = (wsem0, wsem1)

    idx_desc = pltpu.async_copy(idx_hbm.at[pl.ds(j0, JW)], idxvm, rsem)
    x_desc = pltpu.async_copy(x_hbm.at[:, pl.ds(j0, JW)], xbuf, ssem)
    pad_desc = pltpu.async_copy(w_hbm.at[pl.ds(0, 8)], padbuf, ssem)
    idx_desc.wait()

    read_descs = [None] * NCH
    write_descs = [None] * NCH
    read_descs[0] = pltpu.async_copy(
        w_hbm.at[idxvm.at[pl.ds(0, C)]], wbuf.at[0], rsem
    )

    for c in range(NCH):
        buf = c % 2
        read_descs[c].wait()
        if c + 1 < NCH:
            if c >= 1:
                for d in write_descs[c - 1]:
                    d.wait()
                write_descs[c - 1] = None
            read_descs[c + 1] = pltpu.async_copy(
                w_hbm.at[idxvm.at[pl.ds((c + 1) * C, C)]], wbuf.at[1 - buf], rsem
            )
        jc = j0 + c * C
        write_descs[c] = [
            pltpu.async_copy(
                wbuf.at[buf], out_hbm.at[b, pl.ds(jc, C)], wsems[buf]
            )
            for b in range(B)
        ]

    for descs in write_descs:
        if descs is not None:
            for d in descs:
                d.wait()
    x_desc.wait()
    pad_desc.wait()

    lane = lax.broadcasted_iota(jnp.int32, (L,), 0)
    gpb = JW // L

    for b in range(B):

        def group_body(g, _, b=b):
            jg = j0 + g * L
            xv = xbuf[b, pl.ds(g * L, L)]
            padv = jnp.where(xv == PAD, 1, 0).astype(jnp.int32)
            npad = jnp.sum(padv)

            @pl.when(npad != 0)
            def _fix():
                pltpu.sync_copy(out_hbm.at[b, pl.ds(jg, L)], tbuf)
                for r in range(L):
                    is_pad = jnp.sum(jnp.where(lane == r, padv, 0))

                    @pl.when(is_pad != 0)
                    def _patch(r=r):
                        def d_body(dd, _):
                            tbuf[r, pl.ds(dd * L, L)] = padbuf[PAD, pl.ds(dd * L, L)]
                            return 0

                        lax.fori_loop(0, D // L, d_body, 0)

                pltpu.sync_copy(tbuf, out_hbm.at[b, pl.ds(jg, L)])

            return 0

        lax.fori_loop(0, gpb, group_body, 0)


def kernel(X, weights):
    idx = jnp.arange(2, S + 2, dtype=jnp.int32)
    return _sinus_embed(X, weights, idx)
